# baseline (speedup 1.0000x reference)
import functools

import jax
import jax.numpy as jnp
from jax.experimental import pallas as pl
from jax.experimental.pallas import tpu as pltpu


def _round_up(x, m):
    return ((x + m - 1) // m) * m


def _cdiv(a, b):
    return (a + b - 1) // b


def _head_kernel(h_ref, w_ref, b_ref, o_ref, acc_ref, *,
                 inv_seq_len, seq_tail, nk):
    k = pl.program_id(1)

    @pl.when(k == 0)
    def _():
        acc_ref[...] = jnp.zeros_like(acc_ref)

    if seq_tail:
        @pl.when(k != nk - 1)
        def _():
            acc_ref[...] += jnp.sum(h_ref[...].astype(jnp.float32), axis=1)

        @pl.when(k == nk - 1)
        def _():
            h = h_ref[...].astype(jnp.float32)
            sidx = jax.lax.broadcasted_iota(jnp.int32, h.shape, 1)
            acc_ref[...] += jnp.sum(jnp.where(sidx < seq_tail, h, 0.0),
                                    axis=1)
    else:
        acc_ref[...] += jnp.sum(h_ref[...].astype(jnp.float32), axis=1)

    @pl.when(k == nk - 1)
    def _():
        pooled = acc_ref[...] * inv_seq_len
        logits = jax.lax.dot_general(
            pooled, w_ref[...].astype(jnp.float32),
            (((1,), (1,)), ((), ())), preferred_element_type=jnp.float32)
        o_ref[...] = logits + b_ref[...].astype(jnp.float32)


def kernel(hidden_state, weight, bias):
    B, S, H = hidden_state.shape
    L = weight.shape[0]
    h_itemsize = jnp.dtype(hidden_state.dtype).itemsize

    TB = min(16, _round_up(B, 8))
    nb = _cdiv(B, TB)
    row_bytes = H * h_itemsize
    TS = max(8, ((6 << 20) // max(1, TB * row_bytes)) // 8 * 8)
    TS = min(TS, _round_up(S, 8))
    nk = _cdiv(S, TS)
    seq_tail = S - (nk - 1) * TS
    if seq_tail == TS:
        seq_tail = 0

    Lp = _round_up(max(L, 1), 128)
    w2 = weight
    b2 = bias.reshape(1, L)
    if Lp != L:
        w2 = jnp.pad(w2, ((0, Lp - L), (0, 0)))
        b2 = jnp.pad(b2, ((0, 0), (0, Lp - L)))

    need = (2 * TB * TS * H * h_itemsize + H * Lp * 4 + 2 * Lp * 4
            + 2 * TB * Lp * 4 + TB * H * 4)
    vmem_limit = int(min(need + (8 << 20), 56 << 20))

    out = pl.pallas_call(
        functools.partial(_head_kernel, inv_seq_len=1.0 / S,
                          seq_tail=int(seq_tail), nk=nk),
        out_shape=jax.ShapeDtypeStruct((nb * TB, Lp), jnp.float32),
        grid_spec=pltpu.PrefetchScalarGridSpec(
            num_scalar_prefetch=0,
            grid=(nb, nk),
            in_specs=[
                pl.BlockSpec((TB, TS, H), lambda b, k: (b, k, 0)),
                pl.BlockSpec((Lp, H), lambda b, k: (0, 0),
                             pipeline_mode=pl.Buffered(1)),
                pl.BlockSpec((1, Lp), lambda b, k: (0, 0),
                             pipeline_mode=pl.Buffered(1)),
            ],
            out_specs=pl.BlockSpec((TB, Lp), lambda b, k: (b, 0)),
            scratch_shapes=[pltpu.VMEM((TB, H), jnp.float32)],
        ),
        compiler_params=pltpu.CompilerParams(
            dimension_semantics=("parallel", "arbitrary"),
            vmem_limit_bytes=vmem_limit),
    )(hidden_state, w2, b2)

    return out[:B, :L]


# speedup vs baseline: 1.0056x; 1.0056x over previous
import functools

import jax
import jax.numpy as jnp
from jax.experimental import pallas as pl
from jax.experimental.pallas import tpu as pltpu


def _round_up(x, m):
    return ((x + m - 1) // m) * m


_TB = 8


def _head_contig_kernel(h_ref, w_ref, b_ref, o_ref, acc_ref, *,
                        inv_seq_len, seq_len):
    k = pl.program_id(1)
    half = _TB // 2
    h = h_ref[0, 0].astype(jnp.float32)
    s = jnp.sum(h.reshape(half, seq_len, h.shape[-1]), axis=1)

    @pl.when(k == 0)
    def _():
        acc_ref[:half, :] = s

    @pl.when(k == 1)
    def _():
        acc_ref[half:, :] = s
        pooled = acc_ref[...] * inv_seq_len
        logits = jax.lax.dot_general(
            pooled, w_ref[...].astype(jnp.float32),
            (((1,), (1,)), ((), ())), preferred_element_type=jnp.float32)
        o_ref[...] = logits + b_ref[...].astype(jnp.float32)


def kernel(hidden_state, weight, bias):
    B, S, H = hidden_state.shape
    L = weight.shape[0]
    h_itemsize = jnp.dtype(hidden_state.dtype).itemsize

    Lp = _round_up(max(L, 1), 128)
    w2 = weight
    b2 = bias.reshape(1, L)
    if Lp != L:
        w2 = jnp.pad(w2, ((0, Lp - L), (0, 0)))
        b2 = jnp.pad(b2, ((0, 0), (0, Lp - L)))
    fixed = H * Lp * 4 + 2 * Lp * 4 + 2 * _TB * Lp * 4 + _TB * H * 4

    nb = B // _TB
    C = (_TB // 2) * S
    hv = hidden_state.reshape(nb, 2, C, H)
    blk = C * H * h_itemsize
    vmem_limit = int(min(2 * blk + fixed + (8 << 20), 56 << 20))
    out = pl.pallas_call(
        functools.partial(_head_contig_kernel,
                          inv_seq_len=1.0 / S, seq_len=S),
        out_shape=jax.ShapeDtypeStruct((B, Lp), jnp.float32),
        grid_spec=pltpu.PrefetchScalarGridSpec(
            num_scalar_prefetch=0,
            grid=(nb, 2),
            in_specs=[
                pl.BlockSpec((1, 1, C, H), lambda b, k: (b, k, 0, 0)),
                pl.BlockSpec((Lp, H), lambda b, k: (0, 0),
                             pipeline_mode=pl.Buffered(1)),
                pl.BlockSpec((1, Lp), lambda b, k: (0, 0),
                             pipeline_mode=pl.Buffered(1)),
            ],
            out_specs=pl.BlockSpec((_TB, Lp), lambda b, k: (b, 0)),
            scratch_shapes=[pltpu.VMEM((_TB, H), jnp.float32)],
        ),
        compiler_params=pltpu.CompilerParams(
            dimension_semantics=("parallel", "arbitrary"),
            vmem_limit_bytes=vmem_limit),
    )(hv, w2, b2)
    return out[:, :L]
